# baseline (speedup 1.0000x reference)
import functools

import jax
import jax.numpy as jnp
from jax import lax
from jax.experimental import pallas as pl
from jax.experimental.pallas import tpu as pltpu
from jax.experimental.pallas import tpu_sc as plsc

N = 150000
C = 64
S = 8192
NW = 32
NP = 150016
CH = NP // NW
NV = N - (NW - 1) * CH
SEGW = S // NW
LCH = 9376
NBLK = 73
NTAIL = 2
FB = 256
IB = 448
DUMMY = SEGW


def _mesh():
    return plsc.VectorSubcoreMesh(
        core_axis_name="c", subcore_axis_name="s", num_cores=2, num_subcores=16
    )


_PARAMS = pltpu.CompilerParams(
    needs_layout_passes=False, use_tc_tiling_on_sc=False
)


@functools.partial(
    pl.kernel,
    out_type=jax.ShapeDtypeStruct((NP,), jnp.int32),
    mesh=_mesh(),
    compiler_params=_PARAMS,
    scratch_types=[
        pltpu.VMEM((CH * 4,), jnp.int32),
        pltpu.VMEM((CH,), jnp.int32),
    ],
)
def _linearize(coors_hbm, lin_hbm, coors_v, lin_v):
    w = lax.axis_index("c") * 16 + lax.axis_index("s")
    base = w * CH

    @pl.when(w < NW - 1)
    def _():
        pltpu.sync_copy(coors_hbm.at[pl.ds(base * 4, CH * 4)], coors_v)

    @pl.when(w == NW - 1)
    def _():
        pltpu.sync_copy(
            coors_hbm.at[pl.ds((NW - 1) * CH * 4, NV * 4)],
            coors_v.at[pl.ds(0, NV * 4)],
        )

    def body(g, carry):
        flat = g * 64 + lax.iota(jnp.int32, 16) * 4
        b = plsc.load_gather(coors_v, [flat])
        z = plsc.load_gather(coors_v, [flat + 1])
        y = plsc.load_gather(coors_v, [flat + 2])
        x = plsc.load_gather(coors_v, [flat + 3])
        lin = (b << 12) | ((z >> 2) << 8) | ((y >> 2) << 4) | (x >> 2)
        glob = base + g * 16 + lax.iota(jnp.int32, 16)
        lin_v[pl.ds(g * 16, 16)] = jnp.where(glob < N, lin, S)
        return carry

    lax.fori_loop(0, CH // 16, body, 0, unroll=4)
    pltpu.sync_copy(lin_v, lin_hbm.at[pl.ds(base, CH)])


@functools.partial(
    pl.kernel,
    out_type=jax.ShapeDtypeStruct((S, C), jnp.float32),
    mesh=_mesh(),
    compiler_params=_PARAMS,
    scratch_types=[
        pltpu.VMEM((LCH,), jnp.int32),
        pltpu.VMEM((LCH,), jnp.int32),
        pltpu.VMEM((IB,), jnp.int32),
        pltpu.VMEM((FB,), jnp.int32),
        pltpu.VMEM((FB, C), jnp.float32),
        pltpu.VMEM((SEGW + 1, C), jnp.float32),
        pltpu.SemaphoreType.DMA,
        pltpu.SemaphoreType.DMA,
        pltpu.SemaphoreType.DMA,
    ],
)
def _segmax(feat_hbm, lin_hbm, out_hbm, lin_v0, lin_v1, idxbuf, svbuf,
            rows_v, acc_v, gsem, lsem0, lsem1):
    w = lax.axis_index("c") * 16 + lax.axis_index("s")
    lo = w * SEGW
    hi = lo + SEGW
    neginf = jnp.full((16,), -jnp.inf, jnp.float32)
    zero16 = jnp.zeros((16,), jnp.int32)

    def init_acc(i, carry):
        for j in range(C // 16):
            acc_v[i, pl.ds(j * 16, 16)] = neginf
        return carry

    lax.fori_loop(0, SEGW + 1, init_acc, 0)
    dummy16 = zero16 + DUMMY
    for k in range(IB // 16):
        idxbuf[pl.ds(k * 16, 16)] = dummy16

    def issue():
        for k in range(FB // 16):
            pv = idxbuf[pl.ds(k * 16, 16)]
            svbuf[pl.ds(k * 16, 16)] = pv & 511
            pltpu.async_copy(
                feat_hbm.at[pv >> 9], rows_v.at[pl.ds(k * 16, 16)], gsem
            )

    def drain():
        for k in range(FB // 16):
            pltpu.make_async_copy(
                feat_hbm.at[zero16], rows_v.at[pl.ds(k * 16, 16)], gsem
            ).wait()

    def accumulate(n16):
        def blk(bi, carry):
            sv = svbuf[pl.ds(bi * 16, 16)]
            for j in range(16):
                s = sv[j]
                for q in range(C // 16):
                    r = rows_v[bi * 16 + j, pl.ds(q * 16, 16)]
                    a = acc_v[s, pl.ds(q * 16, 16)]
                    acc_v[s, pl.ds(q * 16, 16)] = jnp.maximum(a, r)
            return carry

        lax.fori_loop(0, n16, blk, 0)

    def flush(_n16_unused=None):
        drain()
        accumulate(FB // 16)
        issue()

    def scan_block(buf, q, cur, gi9, nt, check_flush):
        packeds, masks, offs = [], [], [cur]
        for t in range(nt):
            v = buf[pl.ds(q * 128 + t * 16, 16)]
            m = (v >= lo) & (v < hi)
            packeds.append((gi9 + (t << 13)) | (v - lo))
            masks.append(m)
            offs.append(offs[-1] + plsc.all_reduce_population_count(m)[0])
        for t in range(nt):
            plsc.store_compressed(
                idxbuf.at[pl.ds(offs[t], 16)], packeds[t], mask=masks[t]
            )
        cur = offs[nt]
        if check_flush:
            @pl.when(cur >= FB)
            def _():
                flush()
                for k in range((IB - FB) // 16):
                    lv = idxbuf[pl.ds(FB + k * 16, 16)]
                    idxbuf[pl.ds(k * 16, 16)] = lv
            cur = lax.select(cur >= FB, cur - FB, cur)
        return cur

    def scan_chunk(buf, carry):
        cursor, gi8 = carry

        def grp8(q, c):
            cur, g8 = c
            cur = scan_block(buf, q, cur, g8, 8, True)
            return (cur, g8 + (8 << 13))

        cursor, gi8 = lax.fori_loop(0, NBLK, grp8, (cursor, gi8))
        cursor = scan_block(buf, NBLK, cursor, gi8, NTAIL, False)
        return (cursor, gi8 + (NTAIL << 13))

    giota9 = lax.iota(jnp.int32, 16) << 9
    pltpu.async_copy(lin_hbm.at[pl.ds(0, LCH)], lin_v0, lsem0)
    issue()

    def pair(p, carry):
        pltpu.make_async_copy(lin_hbm.at[pl.ds(0, LCH)], lin_v0, lsem0).wait()
        pltpu.async_copy(
            lin_hbm.at[pl.ds((2 * p + 1) * LCH, LCH)], lin_v1, lsem1
        )
        carry = scan_chunk(lin_v0, carry)
        pltpu.make_async_copy(lin_hbm.at[pl.ds(0, LCH)], lin_v1, lsem1).wait()

        @pl.when(p < (NP // LCH) // 2 - 1)
        def _():
            pltpu.async_copy(
                lin_hbm.at[pl.ds((2 * p + 2) * LCH, LCH)], lin_v0, lsem0
            )

        carry = scan_chunk(lin_v1, carry)
        return carry

    cursor, _ = lax.fori_loop(
        0, (NP // LCH) // 2, pair, (jnp.int32(0), giota9)
    )

    @pl.when(cursor >= FB)
    def _():
        flush()
        for k in range((IB - FB) // 16):
            lv = idxbuf[pl.ds(FB + k * 16, 16)]
            idxbuf[pl.ds(k * 16, 16)] = lv

    cursor = lax.select(cursor >= FB, cursor - FB, cursor)
    idxbuf[pl.ds(cursor, 16)] = dummy16
    drain()
    accumulate(FB // 16)
    issue()
    drain()
    accumulate((cursor + 15) >> 4)
    pltpu.sync_copy(
        acc_v.at[pl.ds(0, SEGW)], out_hbm.at[pl.ds(lo, SEGW)]
    )


def kernel(features, coors):
    lin = _linearize(coors.reshape(-1))
    return _segmax(features, lin)


# speedup vs baseline: 1.3304x; 1.3304x over previous
import functools

import jax
import jax.numpy as jnp
from jax import lax
from jax.experimental import pallas as pl
from jax.experimental.pallas import tpu as pltpu
from jax.experimental.pallas import tpu_sc as plsc

N = 150000
C = 64
S = 8192
NW = 32
NP = 150016
CH = NP // NW
NV = N - (NW - 1) * CH
SEGW = S // NW
LCH = 9376
NBLK = 73
NTAIL = 2
FB = 256
IB = 448
DUMMY = SEGW


def _mesh():
    return plsc.VectorSubcoreMesh(
        core_axis_name="c", subcore_axis_name="s", num_cores=2, num_subcores=16
    )


_PARAMS = pltpu.CompilerParams(
    needs_layout_passes=False, use_tc_tiling_on_sc=False
)


CH2 = NP // 16
NV2 = N - 15 * CH2


@functools.partial(
    pl.kernel,
    out_type=jax.ShapeDtypeStruct((2, NP), jnp.int32),
    mesh=_mesh(),
    compiler_params=_PARAMS,
    scratch_types=[
        pltpu.VMEM((CH2 * 4,), jnp.int32),
        pltpu.VMEM((CH2,), jnp.int32),
    ],
)
def _linearize(coors_hbm, lin_hbm, coors_v, lin_v):
    c = lax.axis_index("c")
    s = lax.axis_index("s")
    base = s * CH2

    @pl.when(s < 15)
    def _():
        pltpu.sync_copy(coors_hbm.at[pl.ds(base * 4, CH2 * 4)], coors_v)

    @pl.when(s == 15)
    def _():
        pltpu.sync_copy(
            coors_hbm.at[pl.ds(15 * CH2 * 4, NV2 * 4)],
            coors_v.at[pl.ds(0, NV2 * 4)],
        )

    def body(g, carry):
        flat = g * 64 + lax.iota(jnp.int32, 16) * 4
        b = plsc.load_gather(coors_v, [flat])
        z = plsc.load_gather(coors_v, [flat + 1])
        y = plsc.load_gather(coors_v, [flat + 2])
        x = plsc.load_gather(coors_v, [flat + 3])
        lin = (b << 12) | ((z >> 2) << 8) | ((y >> 2) << 4) | (x >> 2)
        glob = base + g * 16 + lax.iota(jnp.int32, 16)
        lin_v[pl.ds(g * 16, 16)] = jnp.where(glob < N, lin, S)
        return carry

    lax.fori_loop(0, CH2 // 16, body, 0, unroll=4)
    pltpu.sync_copy(lin_v, lin_hbm.at[c, pl.ds(base, CH2)])


@functools.partial(
    pl.kernel,
    out_type=jax.ShapeDtypeStruct((S, C), jnp.float32),
    mesh=_mesh(),
    compiler_params=_PARAMS,
    scratch_types=[
        pltpu.VMEM((LCH,), jnp.int32),
        pltpu.VMEM((LCH,), jnp.int32),
        pltpu.VMEM((IB,), jnp.int32),
        pltpu.VMEM((FB, C), jnp.float32),
        pltpu.VMEM((SEGW + 1, C), jnp.float32),
        pltpu.SemaphoreType.DMA,
        pltpu.SemaphoreType.DMA,
        pltpu.SemaphoreType.DMA,
    ],
)
def _segmax(feat_hbm, lin_hbm, out_hbm, lin_v0, lin_v1, idxbuf, rows_v,
            acc_v, gsem, lsem0, lsem1):
    c = lax.axis_index("c")
    w = c * 16 + lax.axis_index("s")
    lo = w * SEGW
    hi = lo + SEGW
    neginf = jnp.full((16,), -jnp.inf, jnp.float32)
    zero16 = jnp.zeros((16,), jnp.int32)

    def init_acc(i, carry):
        for j in range(C // 16):
            acc_v[i, pl.ds(j * 16, 16)] = neginf
        return carry

    lax.fori_loop(0, SEGW + 1, init_acc, 0)
    dummy16 = zero16 + DUMMY
    for k in range(IB // 16):
        idxbuf[pl.ds(k * 16, 16)] = dummy16

    def accumulate(n16):
        def blk(bi, carry):
            sv = idxbuf[pl.ds(bi * 16, 16)] & 511
            for j in range(16):
                s = sv[j]
                for q in range(C // 16):
                    r = rows_v[bi * 16 + j, pl.ds(q * 16, 16)]
                    a = acc_v[s, pl.ds(q * 16, 16)]
                    acc_v[s, pl.ds(q * 16, 16)] = jnp.maximum(a, r)
            return carry

        lax.fori_loop(0, n16, blk, 0)

    def flush(n16):
        copies = []
        for k in range(FB // 16):
            iv = idxbuf[pl.ds(k * 16, 16)] >> 9
            copies.append(
                pltpu.async_copy(
                    feat_hbm.at[iv], rows_v.at[pl.ds(k * 16, 16)], gsem
                )
            )
        for cp in copies:
            cp.wait()
        accumulate(n16)

    def scan_block(buf, q, cur, gi9, nt, check_flush):
        packeds, masks, offs = [], [], [cur]
        for t in range(nt):
            v = buf[pl.ds(q * 128 + t * 16, 16)]
            m = (v >= lo) & (v < hi)
            packeds.append((gi9 + (t << 13)) | (v - lo))
            masks.append(m)
            offs.append(offs[-1] + plsc.all_reduce_population_count(m)[0])
        for t in range(nt):
            plsc.store_compressed(
                idxbuf.at[pl.ds(offs[t], 16)], packeds[t], mask=masks[t]
            )
        cur = offs[nt]
        if check_flush:
            @pl.when(cur >= FB)
            def _():
                flush(FB // 16)
                for k in range((IB - FB) // 16):
                    lv = idxbuf[pl.ds(FB + k * 16, 16)]
                    idxbuf[pl.ds(k * 16, 16)] = lv
            cur = lax.select(cur >= FB, cur - FB, cur)
        return cur

    def scan_chunk(buf, carry):
        cursor, gi8 = carry

        def grp8(q, c):
            cur, g8 = c
            cur = scan_block(buf, q, cur, g8, 8, True)
            return (cur, g8 + (8 << 13))

        cursor, gi8 = lax.fori_loop(0, NBLK, grp8, (cursor, gi8))
        cursor = scan_block(buf, NBLK, cursor, gi8, NTAIL, False)
        return (cursor, gi8 + (NTAIL << 13))

    giota9 = lax.iota(jnp.int32, 16) << 9
    pltpu.async_copy(lin_hbm.at[c, pl.ds(0, LCH)], lin_v0, lsem0)

    def pair(p, carry):
        pltpu.make_async_copy(lin_hbm.at[c, pl.ds(0, LCH)], lin_v0, lsem0).wait()
        pltpu.async_copy(
            lin_hbm.at[c, pl.ds((2 * p + 1) * LCH, LCH)], lin_v1, lsem1
        )
        carry = scan_chunk(lin_v0, carry)
        pltpu.make_async_copy(lin_hbm.at[c, pl.ds(0, LCH)], lin_v1, lsem1).wait()

        @pl.when(p < (NP // LCH) // 2 - 1)
        def _():
            pltpu.async_copy(
                lin_hbm.at[c, pl.ds((2 * p + 2) * LCH, LCH)], lin_v0, lsem0
            )

        carry = scan_chunk(lin_v1, carry)
        return carry

    cursor, _ = lax.fori_loop(
        0, (NP // LCH) // 2, pair, (jnp.int32(0), giota9)
    )

    @pl.when(cursor >= FB)
    def _():
        flush(FB // 16)
        for k in range((IB - FB) // 16):
            lv = idxbuf[pl.ds(FB + k * 16, 16)]
            idxbuf[pl.ds(k * 16, 16)] = lv

    cursor = lax.select(cursor >= FB, cursor - FB, cursor)
    idxbuf[pl.ds(cursor, 16)] = dummy16
    flush((cursor + 15) >> 4)
    pltpu.sync_copy(
        acc_v.at[pl.ds(0, SEGW)], out_hbm.at[pl.ds(lo, SEGW)]
    )


def kernel(features, coors):
    lin = _linearize(coors.reshape(-1))
    return _segmax(features, lin)
